# baseline (speedup 1.0000x reference)
import math

import jax
import jax.numpy as jnp
from jax.experimental import pallas as pl
from jax.experimental.pallas import tpu as pltpu
from jax.experimental.pallas import tpu_sc as plsc

_PAD = 0
_CONF = 0.9
_N = 2048
_V = 32000
_W = 1280
_GRID = _V // _W
_SLABS = _W // 128
_LAG = 8

_L1 = _CONF
_L0 = (1.0 - _CONF) / (_V - 2)
_C = _L1 * math.log(_L1) + (_V - 1) * _L0 * math.log(_L0)


def _sum_body(m_ref, yp_ref, s_out_ref, npad_ref, s_ref):
    j = pl.program_id(0)

    @pl.when(j == 0)
    def _():
        s_ref[...] = jnp.zeros((_N, 128), jnp.float32)

    part = yp_ref[:, 0:128]
    for c in range(1, _SLABS):
        part = part + yp_ref[:, c * 128:(c + 1) * 128]
    s_ref[...] += part

    @pl.when(j == _GRID - 1)
    def _():
        m = m_ref[...]
        s_out_ref[0, 0] = jnp.sum(s_ref[...] * m)
        npad_ref[0, 0] = jnp.sum(m).astype(jnp.int32)


def _masked_sum(yp, mrow):
    return pl.pallas_call(
        _sum_body,
        grid=(_GRID,),
        in_specs=[
            pl.BlockSpec((_N, 1), lambda j: (0, 0)),
            pl.BlockSpec((_N, _W), lambda j: (0, j)),
        ],
        out_specs=[
            pl.BlockSpec(memory_space=pltpu.SMEM),
            pl.BlockSpec(memory_space=pltpu.SMEM),
        ],
        out_shape=[
            jax.ShapeDtypeStruct((1, 1), jnp.float32),
            jax.ShapeDtypeStruct((1, 1), jnp.int32),
        ],
        scratch_shapes=[pltpu.VMEM((_N, 128), jnp.float32)],
    )(mrow, yp)


def _sc_gather(yp, calign):
    mesh = plsc.VectorSubcoreMesh(
        core_axis_name="core", subcore_axis_name="subcore"
    )
    per_core = 64

    @pl.kernel(
        out_type=jax.ShapeDtypeStruct((8 * _N, 128), jnp.float32),
        mesh=mesh,
        scratch_types=[
            pltpu.VMEM((1, 128), jnp.int32),
            pltpu.SemaphoreType.DMA,
            pltpu.SemaphoreType.DMA,
        ],
    )
    def _k(yp_hbm, c_hbm, o_hbm, idx_vmem, isem, sem):
        core = jax.lax.axis_index("core")
        sub = jax.lax.axis_index("subcore")
        pltpu.async_copy(
            c_hbm.at[:, pl.ds(sub * 128, 128)], idx_vmem, isem
        ).wait()
        base = core * per_core

        @pl.loop(0, per_core)
        def _(i):
            k = base + i
            n = sub * 128 + k
            r0 = (n // 8) * 8
            cvec = idx_vmem[0, pl.ds(k, 1)]
            c = pl.multiple_of(cvec[0], 128)
            pltpu.async_copy(
                yp_hbm.at[pl.ds(r0, 8), pl.ds(c, 128)],
                o_hbm.at[pl.ds(n * 8, 8)],
                sem,
            )

            @pl.when(i >= _LAG)
            def _():
                pltpu.make_async_copy(
                    yp_hbm.at[pl.ds(0, 8), pl.ds(0, 128)],
                    o_hbm.at[pl.ds(0, 8)],
                    sem,
                ).wait()

        @pl.loop(0, _LAG)
        def _(i):
            pltpu.make_async_copy(
                yp_hbm.at[pl.ds(0, 8), pl.ds(0, 128)],
                o_hbm.at[pl.ds(0, 8)],
                sem,
            ).wait()

    return _k(yp, calign.reshape(1, _N))


def _combine_body(g_ref, lane_ref, s_ref, npad_ref, loss_ref, npad_out):
    lanes = jax.lax.broadcasted_iota(jnp.int32, (8 * _N, 128), 1)
    sel = jnp.where(lanes == lane_ref[...], g_ref[...], 0.0)
    gsum = jnp.sum(sel)
    npad_out[0, 0] = npad_ref[0, 0]
    loss_ref[0, 0] = (
        npad_ref[0, 0].astype(jnp.float32) * _C
        - _L0 * s_ref[0, 0]
        - (_L1 - _L0) * gsum
    )


def kernel(y_pred, y_true):
    yp = y_pred.reshape(_N, _V)
    yt = y_true.reshape(_N, 1)
    nonpad = yt != _PAD
    mrow = nonpad.astype(jnp.float32)
    calign = (yt // 128 * 128).reshape(_N)
    sub = jnp.arange(_N, dtype=jnp.int32).reshape(_N, 1) % 8
    sub8 = jax.lax.broadcasted_iota(jnp.int32, (_N, 8), 1)
    lane8 = jnp.where((sub8 == sub) & nonpad, yt % 128, -1)
    lane2 = lane8.reshape(8 * _N, 1)

    s, npad = _masked_sum(yp, mrow)
    g = _sc_gather(yp, calign)

    loss, npad_out = pl.pallas_call(
        _combine_body,
        in_specs=[
            pl.BlockSpec((8 * _N, 128), lambda: (0, 0)),
            pl.BlockSpec((8 * _N, 1), lambda: (0, 0)),
            pl.BlockSpec(memory_space=pltpu.SMEM),
            pl.BlockSpec(memory_space=pltpu.SMEM),
        ],
        out_specs=[
            pl.BlockSpec(memory_space=pltpu.SMEM),
            pl.BlockSpec(memory_space=pltpu.SMEM),
        ],
        out_shape=[
            jax.ShapeDtypeStruct((1, 1), jnp.float32),
            jax.ShapeDtypeStruct((1, 1), jnp.int32),
        ],
    )(g, lane2, s, npad)
    return (loss[0, 0], npad_out[0, 0])


# speedup vs baseline: 3.3841x; 3.3841x over previous
import math

import jax
import jax.numpy as jnp
from jax.experimental import pallas as pl
from jax.experimental.pallas import tpu as pltpu

_PAD = 0
_CONF = 0.9
_N = 2048
_V = 32000
_R = 128
_GRID = _N // _R
_SLABS = _V // 128

_L1 = _CONF
_L0 = (1.0 - _CONF) / (_V - 2)
_C = _L1 * math.log(_L1) + (_V - 1) * _L0 * math.log(_L0)


def _body(yts_ref, m_ref, yp_ref, loss_ref, npad_ref):
    i = pl.program_id(0)
    lane = jax.lax.broadcasted_iota(jnp.int32, (_R, 128), 1)
    d = yts_ref[...] - lane

    part_s = yp_ref[:, 0:128]
    part_g = jnp.where(d == 0, part_s, 0.0)
    for c in range(1, _SLABS):
        slab = yp_ref[:, c * 128:(c + 1) * 128]
        part_s = part_s + slab
        part_g = part_g + jnp.where(d == c * 128, slab, 0.0)

    m = m_ref[...]
    contrib = (
        jnp.sum(m) * _C
        - _L0 * jnp.sum(part_s * m)
        - (_L1 - _L0) * jnp.sum(part_g)
    )
    npad_part = jnp.sum(m).astype(jnp.int32)

    @pl.when(i == 0)
    def _():
        loss_ref[0, 0] = 0.0
        npad_ref[0, 0] = 0

    loss_ref[0, 0] += contrib
    npad_ref[0, 0] += npad_part


def kernel(y_pred, y_true):
    yp = y_pred.reshape(_N, _V)
    yt = y_true.reshape(_N, 1)
    nonpad = yt != _PAD
    yts = jnp.where(nonpad, yt, -1)
    mrow = nonpad.astype(jnp.float32)

    loss, npad = pl.pallas_call(
        _body,
        grid=(_GRID,),
        in_specs=[
            pl.BlockSpec((_R, 1), lambda i: (i, 0)),
            pl.BlockSpec((_R, 1), lambda i: (i, 0)),
            pl.BlockSpec((_R, _V), lambda i: (i, 0)),
        ],
        out_specs=[
            pl.BlockSpec(memory_space=pltpu.SMEM),
            pl.BlockSpec(memory_space=pltpu.SMEM),
        ],
        out_shape=[
            jax.ShapeDtypeStruct((1, 1), jnp.float32),
            jax.ShapeDtypeStruct((1, 1), jnp.int32),
        ],
    )(yts, mrow, yp)
    return (loss[0, 0], npad[0, 0])


# speedup vs baseline: 3.3897x; 1.0017x over previous
import math

import jax
import jax.numpy as jnp
from jax.experimental import pallas as pl
from jax.experimental.pallas import tpu as pltpu

_PAD = 0
_CONF = 0.9
_N = 2048
_V = 32000
_R = 64
_GRID = _N // _R
_SLABS = _V // 128

_L1 = _CONF
_L0 = (1.0 - _CONF) / (_V - 2)
_C = _L1 * math.log(_L1) + (_V - 1) * _L0 * math.log(_L0)


def _body(yts_ref, m_ref, yp_ref, loss_ref, npad_ref):
    i = pl.program_id(0)
    lane = jax.lax.broadcasted_iota(jnp.int32, (_R, 128), 1)
    d = yts_ref[...] - lane

    part_s = yp_ref[:, 0:128]
    part_g = jnp.where(d == 0, part_s, 0.0)
    for c in range(1, _SLABS):
        slab = yp_ref[:, c * 128:(c + 1) * 128]
        part_s = part_s + slab
        part_g = part_g + jnp.where(d == c * 128, slab, 0.0)

    m = m_ref[...]
    contrib = (
        jnp.sum(m) * _C
        - _L0 * jnp.sum(part_s * m)
        - (_L1 - _L0) * jnp.sum(part_g)
    )
    npad_part = jnp.sum(m).astype(jnp.int32)

    @pl.when(i == 0)
    def _():
        loss_ref[0, 0] = 0.0
        npad_ref[0, 0] = 0

    loss_ref[0, 0] += contrib
    npad_ref[0, 0] += npad_part


def kernel(y_pred, y_true):
    yp = y_pred.reshape(_N, _V)
    yt = y_true.reshape(_N, 1)
    nonpad = yt != _PAD
    yts = jnp.where(nonpad, yt, -1)
    mrow = nonpad.astype(jnp.float32)

    loss, npad = pl.pallas_call(
        _body,
        grid=(_GRID,),
        in_specs=[
            pl.BlockSpec((_R, 1), lambda i: (i, 0)),
            pl.BlockSpec((_R, 1), lambda i: (i, 0)),
            pl.BlockSpec((_R, _V), lambda i: (i, 0)),
        ],
        out_specs=[
            pl.BlockSpec(memory_space=pltpu.SMEM),
            pl.BlockSpec(memory_space=pltpu.SMEM),
        ],
        out_shape=[
            jax.ShapeDtypeStruct((1, 1), jnp.float32),
            jax.ShapeDtypeStruct((1, 1), jnp.int32),
        ],
    )(yts, mrow, yp)
    return (loss[0, 0], npad[0, 0])
